# baseline (speedup 1.0000x reference)
import jax
import jax.numpy as jnp
from jax import lax
from jax.experimental import pallas as pl
from jax.experimental.pallas import tpu as pltpu
from jax.experimental.pallas import tpu_sc as plsc
from jax.scipy.linalg import block_diag

B, T, N, C, H, E = 8, 8, 1000, 64, 64, 16000
NG = B * T
GP = 4
W4 = GP * H
NROW = 1024
NSUB = 16
FLAT_N = NROW * NROW
CH = 4096
NCH = FLAT_N // (NSUB * CH)
EPAD = 16384
EROWS = EPAD // 128
SENT = 1000 * NROW


def _sc_build_counts(idx_hbm, out_hbm, idx_v, ones_v, zbuf, shared):
    c = lax.axis_index("c")
    s = lax.axis_index("s")
    off = s * (NCH * CH)

    @pl.when(c == 0)
    def _prep():
        def fz(i, _):
            zbuf[pl.ds(i * 16, 16)] = jnp.zeros((16,), jnp.float32)
            return _
        lax.fori_loop(0, CH // 16, fz, None)
        for k in range(8):
            ones_v[pl.ds(k * 16, 16)] = jnp.full((16,), 1.0, jnp.float32)

        def zc(i, _):
            pltpu.sync_copy(zbuf, shared.at[pl.ds(off + i * CH, CH)])
            return _
        lax.fori_loop(0, NCH, zc, None)
        pltpu.sync_copy(idx_hbm.at[pl.ds(s * 8, 8)], idx_v)

    plsc.subcore_barrier()

    @pl.when(c == 0)
    def _scatter():
        for j in range(8):
            pltpu.sync_copy(ones_v, shared.at[idx_v.at[j]], add=True)

    plsc.subcore_barrier()

    @pl.when(c == 0)
    def _writeback():
        def wb(i, _):
            pltpu.sync_copy(shared.at[pl.ds(off + i * CH, CH)], zbuf)
            pltpu.sync_copy(zbuf, out_hbm.at[pl.ds(off + i * CH, CH)])
            return _
        lax.fori_loop(0, NCH, wb, None)


def _build_counts(flat_idx):
    mesh = plsc.VectorSubcoreMesh(core_axis_name="c", subcore_axis_name="s")
    return pl.kernel(
        _sc_build_counts,
        out_type=jax.ShapeDtypeStruct((FLAT_N,), jnp.float32),
        mesh=mesh,
        scratch_types=[
            pltpu.VMEM((8, 128), jnp.int32),
            pltpu.VMEM((128,), jnp.float32),
            pltpu.VMEM((CH,), jnp.float32),
            pltpu.VMEM_SHARED((FLAT_N,), jnp.float32),
        ],
    )(flat_idx)


def _tc_forward(acnt_ref, x_ref, ws_in, b_in, g_in, bb_in, w_h0, b_h0, g_h0,
                bb_h0, w_h1, b_h1, g_h1, bb_h1, w_c1, b_c1, w_c2, b_c2,
                m4_ref, fold_ref, out_ref, a_scr, dinv_ref, acc_ref):
    g4 = pl.program_id(0)

    @pl.when(g4 == 0)
    def _init():
        for j in range(NROW // 128):
            a_scr[:, pl.ds(j * 128, 128)] = acnt_ref[:, j, :]
        deg = jnp.sum(a_scr[...], axis=1, keepdims=True) + 1.0
        dinv_ref[...] = lax.rsqrt(deg)
        acc_ref[...] = jnp.zeros_like(acc_ref)

    dinv = dinv_ref[...]
    a = a_scr[...]
    m4 = m4_ref[...]

    def conv(h_in, w, b):
        h = jnp.dot(h_in, w[...], preferred_element_type=jnp.float32)
        hs = h * dinv
        agg = jnp.dot(a, hs, preferred_element_type=jnp.float32)
        return dinv * (agg + hs) + b[...]

    def lnorm(z, gg, bb):
        m = jnp.dot(z, m4, preferred_element_type=jnp.float32)
        zc = z - m
        v = jnp.dot(zc * zc, m4, preferred_element_type=jnp.float32)
        return zc * lax.rsqrt(v + 1e-5) * gg[...] + bb[...]

    h0 = jnp.dot(x_ref[0], ws_in[0], preferred_element_type=jnp.float32)
    for g in range(1, GP):
        h0 = h0 + jnp.dot(x_ref[g], ws_in[g],
                          preferred_element_type=jnp.float32)
    h0 = jnp.concatenate(
        [h0, jnp.zeros((NROW - N, W4), jnp.float32)], axis=0)
    hs0 = h0 * dinv
    z0 = dinv * (jnp.dot(a, hs0, preferred_element_type=jnp.float32)
                 + hs0) + b_in[...]
    out = jax.nn.relu(lnorm(z0, g_in, bb_in))
    out = jax.nn.relu(lnorm(conv(out, w_h0, b_h0), g_h0, bb_h0)) + out
    out = jax.nn.relu(lnorm(conv(out, w_h1, b_h1), g_h1, bb_h1)) + out

    pooled_row = jnp.sum(out[:N], axis=0, keepdims=True) * (1.0 / (N * T))
    b_idx = g4 // (T // GP)
    acc_ref[pl.ds(b_idx, 1), :] += pooled_row

    @pl.when(g4 == NG // GP - 1)
    def _head():
        pooled = jnp.dot(acc_ref[...], fold_ref[...],
                         preferred_element_type=jnp.float32)
        hh = jax.nn.relu(
            jnp.dot(pooled, w_c1[...],
                    preferred_element_type=jnp.float32) + b_c1[...])
        out_ref[...] = (jnp.dot(hh, w_c2[...],
                                preferred_element_type=jnp.float32)
                        + b_c2[...])


def _forward(acnt_pad, xcols, W_in, b_in, ln_in_g, ln_in_b, W_h0, b_h0,
             ln_h0_g, ln_h0_b, W_h1, b_h1, ln_h1_g, ln_h1_b,
             W_c1, b_c1, W_c2, b_c2):
    full = lambda shape: pl.BlockSpec(shape, lambda g: (0,) * len(shape))
    bd4 = lambda w: block_diag(w, w, w, w)
    row4 = lambda v: jnp.tile(v, GP).reshape(1, W4)
    m4 = block_diag(*([jnp.full((H, H), 1.0 / H, jnp.float32)] * GP))
    fold = jnp.tile(jnp.eye(H, dtype=jnp.float32), (GP, 1))
    ws_in = bd4(W_in).reshape(GP, C, W4)
    return pl.pallas_call(
        _tc_forward,
        grid=(NG // GP,),
        in_specs=[
            full((NROW, NROW // 128, 128)),
            pl.BlockSpec((GP, N, C), lambda g: (g, 0, 0)),
            full((GP, C, W4)), full((1, W4)), full((1, W4)), full((1, W4)),
            full((W4, W4)), full((1, W4)), full((1, W4)), full((1, W4)),
            full((W4, W4)), full((1, W4)), full((1, W4)), full((1, W4)),
            full((H, H)), full((1, H)), full((H, 2)), full((1, 2)),
            full((W4, W4)), full((W4, H)),
        ],
        out_specs=full((B, 2)),
        out_shape=jax.ShapeDtypeStruct((B, 2), jnp.float32),
        scratch_shapes=[
            pltpu.VMEM((NROW, NROW), jnp.float32),
            pltpu.VMEM((NROW, 1), jnp.float32),
            pltpu.VMEM((B, W4), jnp.float32),
        ],
    )(acnt_pad, xcols, ws_in, row4(b_in), row4(ln_in_g), row4(ln_in_b),
      bd4(W_h0), row4(b_h0), row4(ln_h0_g), row4(ln_h0_b),
      bd4(W_h1), row4(b_h1), row4(ln_h1_g), row4(ln_h1_b),
      W_c1, b_c1.reshape(1, H), W_c2, b_c2.reshape(1, 2), m4, fold)


def kernel(x, base_adj, base_edge_index, W_in, b_in, ln_in_g, ln_in_b,
           W_h0, b_h0, ln_h0_g, ln_h0_b, W_h1, b_h1, ln_h1_g, ln_h1_b,
           W_c1, b_c1, W_c2, b_c2):
    src = base_edge_index[0]
    dst = base_edge_index[1]
    flat = dst * NROW + src
    flat = jnp.concatenate(
        [flat, jnp.full((EPAD - E,), SENT, jnp.int32)]).reshape(EROWS, 128)
    counts = _build_counts(flat)
    acnt_pad = counts.reshape(NROW, NROW // 128, 128)
    xcols = x.reshape(NG, N, C)
    return _forward(acnt_pad, xcols, W_in, b_in, ln_in_g, ln_in_b, W_h0, b_h0,
                    ln_h0_g, ln_h0_b, W_h1, b_h1, ln_h1_g, ln_h1_b,
                    W_c1, b_c1, W_c2, b_c2)


# speedup vs baseline: 1.1143x; 1.1143x over previous
import jax
import jax.numpy as jnp
from jax import lax
from jax.experimental import pallas as pl
from jax.experimental.pallas import tpu as pltpu
from jax.experimental.pallas import tpu_sc as plsc
from jax.scipy.linalg import block_diag

B, T, N, C, H, E = 8, 8, 1000, 64, 64, 16000
NG = B * T
GP = 4
W4 = GP * H
NROW = 1024
NSUB = 16
FLAT_N = NROW * NROW
CH = 4096
NCH = FLAT_N // (NSUB * CH)
EPAD = 16384
EROWS = EPAD // 128
SENT = 1000 * NROW


def _sc_build_counts(idx_hbm, out_hbm, idx_v, ones_v, zbuf, shared):
    c = lax.axis_index("c")
    s = lax.axis_index("s")
    off = s * (NCH * CH)

    @pl.when(c == 0)
    def _prep():
        def fz(i, _):
            zbuf[pl.ds(i * 16, 16)] = jnp.zeros((16,), jnp.float32)
            return _
        lax.fori_loop(0, CH // 16, fz, None)
        for k in range(8):
            ones_v[pl.ds(k * 16, 16)] = jnp.full((16,), 1.0, jnp.float32)

        def zc(i, _):
            pltpu.sync_copy(zbuf, shared.at[pl.ds(off + i * CH, CH)])
            return _
        lax.fori_loop(0, NCH, zc, None)
        pltpu.sync_copy(idx_hbm.at[pl.ds(s * 8, 8)], idx_v)

    plsc.subcore_barrier()

    @pl.when(c == 0)
    def _scatter():
        for j in range(8):
            pltpu.sync_copy(ones_v, shared.at[idx_v.at[j]], add=True)

    plsc.subcore_barrier()

    @pl.when(c == 0)
    def _writeback():
        def wb(i, _):
            pltpu.sync_copy(shared.at[pl.ds(off + i * CH, CH)], zbuf)
            pltpu.sync_copy(zbuf, out_hbm.at[pl.ds(off + i * CH, CH)])
            return _
        lax.fori_loop(0, NCH, wb, None)


def _build_counts(flat_idx):
    mesh = plsc.VectorSubcoreMesh(core_axis_name="c", subcore_axis_name="s")
    return pl.kernel(
        _sc_build_counts,
        out_type=jax.ShapeDtypeStruct((FLAT_N,), jnp.float32),
        mesh=mesh,
        scratch_types=[
            pltpu.VMEM((8, 128), jnp.int32),
            pltpu.VMEM((128,), jnp.float32),
            pltpu.VMEM((CH,), jnp.float32),
            pltpu.VMEM_SHARED((FLAT_N,), jnp.float32),
        ],
    )(flat_idx)


def _tc_forward(acnt_ref, x_ref, ws_in, b_in, g_in, bb_in, w_h0, b_h0, g_h0,
                bb_h0, w_h1, b_h1, g_h1, bb_h1, w_c1, b_c1, w_c2, b_c2,
                m4_ref, fold_ref, out_ref, a_scr, dinv_ref, acc_ref):
    g4 = pl.program_id(0)

    @pl.when(g4 == 0)
    def _init():
        for j in range(NROW // 128):
            a_scr[:, pl.ds(j * 128, 128)] = acnt_ref[:, j, :]
        deg = jnp.sum(a_scr[...], axis=1, keepdims=True) + 1.0
        dinv_ref[...] = lax.rsqrt(deg)
        acc_ref[...] = jnp.zeros_like(acc_ref)

    dinv = dinv_ref[...]
    a = a_scr[...]
    m4 = m4_ref[...]

    def conv(h_in, w, b):
        h = jnp.dot(h_in, w[...], preferred_element_type=jnp.float32)
        hs = h * dinv
        agg = jnp.dot(a, hs, preferred_element_type=jnp.float32)
        return dinv * (agg + hs) + b[...]

    def lnorm(z, gg, bb):
        m = jnp.dot(z, m4, preferred_element_type=jnp.float32)
        zc = z - m
        v = jnp.dot(zc * zc, m4, preferred_element_type=jnp.float32)
        return zc * lax.rsqrt(v + 1e-5) * gg[...] + bb[...]

    tn = (((0,), (0,)), ((), ()))
    h0 = lax.dot_general(x_ref[0], ws_in[0], tn,
                         preferred_element_type=jnp.float32)
    for g in range(1, GP):
        h0 = h0 + lax.dot_general(x_ref[g], ws_in[g], tn,
                                  preferred_element_type=jnp.float32)
    h0 = jnp.concatenate(
        [h0, jnp.zeros((NROW - N, W4), jnp.float32)], axis=0)
    hs0 = h0 * dinv
    z0 = dinv * (jnp.dot(a, hs0, preferred_element_type=jnp.float32)
                 + hs0) + b_in[...]
    out = jax.nn.relu(lnorm(z0, g_in, bb_in))
    out = jax.nn.relu(lnorm(conv(out, w_h0, b_h0), g_h0, bb_h0)) + out
    out = jax.nn.relu(lnorm(conv(out, w_h1, b_h1), g_h1, bb_h1)) + out

    pooled_row = jnp.sum(out[:N], axis=0, keepdims=True) * (1.0 / (N * T))
    b_idx = g4 // (T // GP)
    acc_ref[pl.ds(b_idx, 1), :] += pooled_row

    @pl.when(g4 == NG // GP - 1)
    def _head():
        pooled = jnp.dot(acc_ref[...], fold_ref[...],
                         preferred_element_type=jnp.float32)
        hh = jax.nn.relu(
            jnp.dot(pooled, w_c1[...],
                    preferred_element_type=jnp.float32) + b_c1[...])
        out_ref[...] = (jnp.dot(hh, w_c2[...],
                                preferred_element_type=jnp.float32)
                        + b_c2[...])


def _forward(acnt_pad, xcols, W_in, b_in, ln_in_g, ln_in_b, W_h0, b_h0,
             ln_h0_g, ln_h0_b, W_h1, b_h1, ln_h1_g, ln_h1_b,
             W_c1, b_c1, W_c2, b_c2):
    full = lambda shape: pl.BlockSpec(shape, lambda g: (0,) * len(shape))
    bd4 = lambda w: block_diag(w, w, w, w)
    row4 = lambda v: jnp.tile(v, GP).reshape(1, W4)
    m4 = block_diag(*([jnp.full((H, H), 1.0 / H, jnp.float32)] * GP))
    fold = jnp.tile(jnp.eye(H, dtype=jnp.float32), (GP, 1))
    ws_in = bd4(W_in).reshape(GP, C, W4)
    return pl.pallas_call(
        _tc_forward,
        grid=(NG // GP,),
        in_specs=[
            full((NROW, NROW // 128, 128)),
            pl.BlockSpec((GP, C, N), lambda g: (g, 0, 0)),
            full((GP, C, W4)), full((1, W4)), full((1, W4)), full((1, W4)),
            full((W4, W4)), full((1, W4)), full((1, W4)), full((1, W4)),
            full((W4, W4)), full((1, W4)), full((1, W4)), full((1, W4)),
            full((H, H)), full((1, H)), full((H, 2)), full((1, 2)),
            full((W4, W4)), full((W4, H)),
        ],
        out_specs=full((B, 2)),
        out_shape=jax.ShapeDtypeStruct((B, 2), jnp.float32),
        scratch_shapes=[
            pltpu.VMEM((NROW, NROW), jnp.float32),
            pltpu.VMEM((NROW, 1), jnp.float32),
            pltpu.VMEM((B, W4), jnp.float32),
        ],
    )(acnt_pad, xcols, ws_in, row4(b_in), row4(ln_in_g), row4(ln_in_b),
      bd4(W_h0), row4(b_h0), row4(ln_h0_g), row4(ln_h0_b),
      bd4(W_h1), row4(b_h1), row4(ln_h1_g), row4(ln_h1_b),
      W_c1, b_c1.reshape(1, H), W_c2, b_c2.reshape(1, 2), m4, fold)


def kernel(x, base_adj, base_edge_index, W_in, b_in, ln_in_g, ln_in_b,
           W_h0, b_h0, ln_h0_g, ln_h0_b, W_h1, b_h1, ln_h1_g, ln_h1_b,
           W_c1, b_c1, W_c2, b_c2):
    src = base_edge_index[0]
    dst = base_edge_index[1]
    flat = dst * NROW + src
    flat = jnp.concatenate(
        [flat, jnp.full((EPAD - E,), SENT, jnp.int32)]).reshape(EROWS, 128)
    counts = _build_counts(flat)
    acnt_pad = counts.reshape(NROW, NROW // 128, 128)
    xcols = x.transpose(0, 1, 3, 2).reshape(NG, C, N)
    return _forward(acnt_pad, xcols, W_in, b_in, ln_in_g, ln_in_b, W_h0, b_h0,
                    ln_h0_g, ln_h0_b, W_h1, b_h1, ln_h1_g, ln_h1_b,
                    W_c1, b_c1, W_c2, b_c2)
